# Initial kernel scaffold; baseline (speedup 1.0000x reference)
import jax
import jax.numpy as jnp
from jax.experimental import pallas as pl


def kernel(x, adj0, adj1, bipartite, argadj, params):
    raise NotImplementedError("write your pallas kernel here")


# speedup vs baseline: 1.3923x; 1.3923x over previous
import functools

import jax
import jax.numpy as jnp
from jax import lax
from jax.experimental import pallas as pl
from jax.experimental.pallas import tpu as pltpu
from jax.experimental.pallas import tpu_sc as plsc

N0, N1 = 10242, 2562
N0P, N1P = 10752, 3072
KA, KB = 7, 16
NC, NS, NW = 2, 16, 32
SLOPE = 0.01


def _leaky(x):
    return jnp.maximum(x, SLOPE * x)


def _gwidth(cdim):
    return ((cdim + 16) + 127) // 128 * 128


def _mm_body(gw, ones, q_ref, w_ref, a_ref, g_ref, e_ref):
    h = jnp.dot(q_ref[...], w_ref[...], preferred_element_type=jnp.float32)
    e = jnp.dot(h, a_ref[...], preferred_element_type=jnp.float32)
    bn = h.shape[0]
    parts = [h, e, jnp.zeros((bn, 8), jnp.float32)]
    if ones:
        parts.append(jnp.ones((bn, 16), jnp.float32))
    w = sum(x.shape[1] for x in parts)
    parts.append(jnp.zeros((bn, gw - w), jnp.float32))
    g_ref[...] = jnp.concatenate(parts, axis=1)
    e_ref[...] = e


def _mm(q, wt, a, bn=384, ones=False):
    n, qf = q.shape
    cdim = wt.shape[1]
    gw = _gwidth(cdim)
    return pl.pallas_call(
        functools.partial(_mm_body, gw, ones),
        grid=(n // bn,),
        in_specs=[
            pl.BlockSpec((bn, qf), lambda i: (i, 0)),
            pl.BlockSpec((qf, cdim), lambda i: (0, 0)),
            pl.BlockSpec((cdim, 8), lambda i: (0, 0)),
        ],
        out_specs=[
            pl.BlockSpec((bn, gw), lambda i: (i, 0)),
            pl.BlockSpec((bn, 8), lambda i: (i, 0)),
        ],
        out_shape=[
            jax.ShapeDtypeStruct((n, gw), jnp.float32),
            jax.ShapeDtypeStruct((n, 8), jnp.float32),
        ],
    )(q, wt, a)


def _combine_body(p_ref, g_ref, e_ref):
    x = p_ref[...]
    s = x[:10] + x[10:]
    cat = jnp.concatenate([s[j] for j in range(10)], axis=1)
    cnt = jnp.maximum(cat[:, 144:145], 1.0)
    gi = cat[:, :136] / cnt
    bn = gi.shape[0]
    g_ref[...] = jnp.concatenate([gi, jnp.zeros((bn, 120), jnp.float32)],
                                 axis=1)
    e_ref[...] = gi[:, 128:136]


def _combine(p20, bn=384):
    return pl.pallas_call(
        _combine_body,
        grid=(N1P // bn,),
        in_specs=[pl.BlockSpec((20, bn, 16), lambda i: (0, i, 0))],
        out_specs=[
            pl.BlockSpec((bn, 256), lambda i: (i, 0)),
            pl.BlockSpec((bn, 8), lambda i: (i, 0)),
        ],
        out_shape=[
            jax.ShapeDtypeStruct((N1P, 256), jnp.float32),
            jax.ShapeDtypeStruct((N1P, 8), jnp.float32),
        ],
    )(p20)


def _agg(gsrc, eqd, adjf, *, cdim, heads, k, ndst, ew, leaky, ones_col):
    gw = gsrc.shape[1]
    ow = gw if ones_col else cdim
    rpt = ndst // NW
    ngrp = rpt // 16
    o = cdim // heads
    rows = 16 * k
    nch = -(-rows // 128)
    ch = rows // nch
    mesh = plsc.VectorSubcoreMesh(core_axis_name="c", subcore_axis_name="s")

    def body(gsrc_ref, eqd_flat, adj_ref, out_ref,
             idx_v, eqv, gbuf, obuf, abuf, sem):
        wid = lax.axis_index("s") * NC + lax.axis_index("c")
        base = wid * rpt
        pltpu.sync_copy(adj_ref.at[pl.ds(base * k, rpt * k)], idx_v)
        pltpu.sync_copy(eqd_flat.at[pl.ds(base * ew, rpt * ew)], eqv)
        iota = lax.iota(jnp.int32, 16)
        if ones_col:
            for n in range(16):
                for j in range(cdim + 16, gw, 16):
                    obuf[n, pl.ds(j, 16)] = jnp.zeros((16,), jnp.float32)

        def group(g, carry):
            for j in range(nch):
                pltpu.async_copy(
                    gsrc_ref.at[idx_v.at[pl.ds(g * rows + j * ch, ch)]],
                    gbuf.at[pl.ds(j * ch, ch)], sem).wait()
            for h in range(heads):
                eq = plsc.load_gather(
                    eqv, [(g * 16 + iota) * ew + h])
                ls = []
                for kk in range(k):
                    ek = plsc.load_gather(
                        gbuf, [iota * k + kk,
                               jnp.full((16,), cdim + 4 + h, jnp.int32)])
                    ls.append(_leaky(eq + ek))
                m = ls[0]
                for kk in range(1, k):
                    m = jnp.maximum(m, ls[kk])
                es = [jnp.exp(l - m) for l in ls]
                s = es[0]
                for kk in range(1, k):
                    s = s + es[kk]
                r = 1.0 / s
                for kk in range(k):
                    abuf[pl.ds((h * k + kk) * 16, 16)] = es[kk] * r

            def node(n, c2):
                nv = jnp.full((16,), n, jnp.int32)
                for h in range(heads):
                    avs = [plsc.load_gather(
                        abuf, [jnp.full((16,), (h * k + kk) * 16, jnp.int32)
                               + nv])
                        for kk in range(k)]
                    for ob in range(o // 16):
                        cb = h * o + ob * 16
                        acc = jnp.zeros((16,), jnp.float32)
                        for kk in range(k):
                            acc = acc + avs[kk] * gbuf[n * k + kk, pl.ds(cb, 16)]
                        if leaky:
                            acc = _leaky(acc)
                        obuf[n, pl.ds(cb, 16)] = acc
                if ones_col:
                    obuf[n, pl.ds(cdim, 16)] = jnp.where(iota == 0, 1.0, 0.0)
                return c2
            lax.fori_loop(0, 16, node, 0)
            pltpu.sync_copy(obuf, out_ref.at[pl.ds(base + g * 16, 16)])
            return carry
        lax.fori_loop(0, ngrp, group, 0)

    fn = pl.kernel(
        body,
        out_type=jax.ShapeDtypeStruct((ndst, ow), jnp.float32),
        mesh=mesh,
        compiler_params=pltpu.CompilerParams(needs_layout_passes=False),
        scratch_types=[
            pltpu.VMEM((rpt * k,), jnp.int32),
            pltpu.VMEM((rpt * ew,), jnp.float32),
            pltpu.VMEM((rows, gw), jnp.float32),
            pltpu.VMEM((16, ow), jnp.float32),
            pltpu.VMEM((heads * k * 16,), jnp.float32),
            pltpu.SemaphoreType.DMA,
        ],
    )
    return fn(gsrc, eqd.reshape(-1), adjf)


def _pool(gk, argp):
    half = N0P // 2
    chunk = 112
    nchunk = half // chunk
    mesh = plsc.VectorSubcoreMesh(core_axis_name="c", subcore_axis_name="s")

    def body(g_ref, a_ref, out_ref, qv, acc, idxv):
        wid = lax.axis_index("s") * NC + lax.axis_index("c")

        def zrow(i, c2):
            acc[pl.ds(pl.multiple_of(i * 16, 16), 16)] = (
                jnp.zeros((16,), jnp.float32))
            return c2
        lax.fori_loop(0, N1P, zrow, 0)

        @pl.when(wid < 20)
        def _():
            halfsel = wid // 10
            block = wid - 10 * halfsel
            sup = pl.multiple_of(jnp.where(block < 8, 0, 128), 128)
            off = pl.multiple_of((block * 16) % 128, 16)

            def do_chunk(ck, c2):
                row0 = halfsel * half + ck * chunk
                pltpu.sync_copy(g_ref.at[pl.ds(row0, chunk),
                                         pl.ds(sup, 128)], qv)
                pltpu.sync_copy(a_ref.at[pl.ds(row0, chunk)],
                                idxv.at[pl.ds(0, chunk)])

                def srow(j, c3):
                    mv = idxv[pl.ds(j, 16)]
                    m = pl.ds(pl.multiple_of(mv[0] * 16, 16), 16)
                    acc[m] = acc[m] + qv[j, pl.ds(off, 16)]
                    return c3
                lax.fori_loop(0, chunk, srow, 0)
                return c2
            lax.fori_loop(0, nchunk, do_chunk, 0)
            pltpu.sync_copy(acc, out_ref.at[wid])


    fn = pl.kernel(
        body,
        out_type=jax.ShapeDtypeStruct((20, N1P * 16), jnp.float32),
        mesh=mesh,
        compiler_params=pltpu.CompilerParams(needs_layout_passes=False),
        scratch_types=[
            pltpu.VMEM((chunk, 128), jnp.float32),
            pltpu.VMEM((N1P * 16,), jnp.float32),
            pltpu.VMEM((chunk + 16,), jnp.int32),
        ],
    )
    return fn(gk, argp)


def _gadd(gtop, gp, arg3):
    rpt = N0P // NW
    mesh = plsc.VectorSubcoreMesh(core_axis_name="c", subcore_axis_name="s")

    def body(gt_ref, gp_ref, a_ref, out_ref, e_ref, tv, pv, ev, idxv, sem):
        wid = lax.axis_index("s") * NC + lax.axis_index("c")
        base = wid * rpt
        pltpu.sync_copy(a_ref.at[wid], idxv)
        for j in range(3):
            pltpu.async_copy(gp_ref.at[idxv.at[j]], pv, sem).wait()
            pltpu.sync_copy(gt_ref.at[pl.ds(base + j * 112, 112)], tv)

            def row(i, c2):
                for c in range(256 // 16):
                    sl = pl.ds(c * 16, 16)
                    tv[i, sl] = tv[i, sl] + pv[i, sl]
                ev[i, :] = tv[i, pl.ds(128, 16)]
                return c2
            lax.fori_loop(0, 112, row, 0)
            pltpu.sync_copy(tv, out_ref.at[pl.ds(base + j * 112, 112)])
            pltpu.sync_copy(ev, e_ref.at[pl.ds(base + j * 112, 112)])

    fn = pl.kernel(
        body,
        out_type=[
            jax.ShapeDtypeStruct((N0P, 256), jnp.float32),
            jax.ShapeDtypeStruct((N0P, 16), jnp.float32),
        ],
        mesh=mesh,
        compiler_params=pltpu.CompilerParams(needs_layout_passes=False),
        scratch_types=[
            pltpu.VMEM((112, 256), jnp.float32),
            pltpu.VMEM((112, 256), jnp.float32),
            pltpu.VMEM((112, 16), jnp.float32),
            pltpu.VMEM((3, 112), jnp.int32),
            pltpu.SemaphoreType.DMA,
        ],
    )
    return fn(gtop, gp, arg3)


def _wmat(p):
    w = p["W"]
    h, o, qf = w.shape
    wt = w.reshape(h * o, qf).T
    aqf = p["aq"].reshape(h * o)
    akf = p["ak"].reshape(h * o)
    headid = jnp.repeat(jnp.arange(h), o)
    eye = (headid[:, None] == jnp.arange(h)[None, :]).astype(jnp.float32)
    a = jnp.zeros((h * o, 8), jnp.float32)
    a = a.at[:, :h].set(aqf[:, None] * eye)
    a = a.at[:, 4:4 + h].set(akf[:, None] * eye)
    return wt, a


def kernel(x, adj0, adj1, bipartite, argadj, params):
    xt = jnp.pad(x.T, ((0, N0P - N0), (0, 0)))
    adj0f = jnp.pad(adj0, ((0, N0P - N0), (0, 0))).reshape(-1)
    adj1f = jnp.pad(adj1, ((0, N1P - N1), (0, 0))).reshape(-1)
    bipf = jnp.pad(bipartite, ((0, N1P - N1), (0, 0))).reshape(-1)
    argp = jnp.pad(argadj, (0, N0P - N0), constant_values=N1P - 1)
    arg3 = argp.reshape(NW, 3, 112)

    p = params
    wt, a = _wmat(p["c0"]["l1"])
    g, e = _mm(xt, wt, a)
    q = _agg(g, e, adj0f, cdim=256, heads=4, k=KA, ndst=N0P, ew=8,
             leaky=True, ones_col=False)

    wt, a = _wmat(p["c0"]["l2"])
    g, e = _mm(q, wt, a)
    q0 = _agg(g, e, adj0f, cdim=256, heads=4, k=KA, ndst=N0P, ew=8,
              leaky=True, ones_col=False)

    wt, a = _wmat(p["r01"])
    gk, _ = _mm(q0, wt, a, ones=True)
    p20 = _pool(gk, argp).reshape(20, N1P, 16)
    gq, eq = _combine(p20)
    q1 = _agg(gk, eq, bipf, cdim=128, heads=4, k=KB, ndst=N1P, ew=8,
              leaky=True, ones_col=False)

    wt, a = _wmat(p["c1"]["l1"])
    g, e = _mm(q1, wt, a)
    q1 = _agg(g, e, adj1f, cdim=512, heads=4, k=KA, ndst=N1P, ew=8,
              leaky=True, ones_col=False)
    wt, a = _wmat(p["c1"]["l2"])
    g, e = _mm(q1, wt, a)
    q1 = _agg(g, e, adj1f, cdim=512, heads=4, k=KA, ndst=N1P, ew=8,
              leaky=True, ones_col=False)

    wt, a = _wmat(p["e1"]["l1"])
    g, e = _mm(q1, wt, a)
    e1 = _agg(g, e, adj1f, cdim=512, heads=4, k=KA, ndst=N1P, ew=8,
              leaky=True, ones_col=False)
    wt, a = _wmat(p["e1"]["l2"])
    g, e = _mm(e1, wt, a)
    e1 = _agg(g, e, adj1f, cdim=512, heads=4, k=KA, ndst=N1P, ew=8,
              leaky=True, ones_col=False)

    wt, a = _wmat(p["e0"]["l1"])
    gtop, _ = _mm(q0, wt[:256], a)
    gp, _ = _mm(e1, wt[256:], a)
    ge0, ee0 = _gadd(gtop, gp, arg3)
    q = _agg(ge0, ee0, adj0f, cdim=128, heads=4, k=KA, ndst=N0P, ew=16,
             leaky=True, ones_col=False)

    wt, a = _wmat(p["e0"]["l2"])
    g, e = _mm(q, wt, a)
    q = _agg(g, e, adj0f, cdim=128, heads=4, k=KA, ndst=N0P, ew=8,
             leaky=True, ones_col=False)

    wt, a = _wmat(p["ro"])
    g, e = _mm(q, wt, a)
    out = _agg(g, e, adj0f, cdim=16, heads=1, k=KA, ndst=N0P, ew=8,
               leaky=False, ones_col=False)
    return out[:N0].T
